# Initial kernel scaffold; baseline (speedup 1.0000x reference)
import jax
import jax.numpy as jnp
from jax.experimental import pallas as pl


def kernel(flat, cu_seqlens):
    raise NotImplementedError("write your pallas kernel here")


# speedup vs baseline: 6.0584x; 6.0584x over previous
import functools

import jax
import jax.numpy as jnp
from jax import lax
from jax.experimental import pallas as pl
from jax.experimental.pallas import tpu as pltpu
from jax.experimental.pallas import tpu_sc as plsc

NC = 2
NS = 16
NW = NC * NS
LANES = 16
CHUNK = 256

NEG = float("-inf")


def _sc_stage1(flat1d, starts, ends, n, d, b):
    rows_w = n // NW
    nchunk = rows_w // CHUNK
    mesh = plsc.VectorSubcoreMesh(
        core_axis_name="c", subcore_axis_name="s", num_cores=NC, num_subcores=NS
    )

    @functools.partial(
        pl.kernel,
        out_type=jax.ShapeDtypeStruct((NW * b * d,), jnp.float32),
        mesh=mesh,
        scratch_types=[
            pltpu.VMEM((CHUNK * d,), jnp.float32),
            pltpu.VMEM((CHUNK * d,), jnp.float32),
            pltpu.VMEM((b,), jnp.int32),
            pltpu.VMEM((b,), jnp.int32),
            pltpu.VMEM((b * d,), jnp.float32),
            pltpu.SemaphoreType.DMA,
            pltpu.SemaphoreType.DMA,
        ],
    )
    def k(flat_hbm, st_hbm, en_hbm, out_hbm, buf0, buf1, st_v, en_v, acc_v, sem0, sem1):
        cid = lax.axis_index("c")
        sid = lax.axis_index("s")
        wid = sid * NC + cid
        base = wid * rows_w * d

        pltpu.sync_copy(st_hbm.at[pl.ds(wid * b, b)], st_v)
        pltpu.sync_copy(en_hbm.at[pl.ds(wid * b, b)], en_v)
        st_vec = st_v[...]
        en_vec = en_v[...]

        neg = jnp.full((LANES,), NEG, jnp.float32)
        for kk in range(b * d // LANES):
            acc_v[pl.ds(kk * LANES, LANES)] = neg

        bufs = (buf0, buf1)
        sems = (sem0, sem1)
        pltpu.make_async_copy(
            flat_hbm.at[pl.ds(base, CHUNK * d)], buf0, sem0
        ).start()
        for c in range(nchunk):
            buf = bufs[c % 2]
            sem = sems[c % 2]
            pltpu.make_async_copy(
                flat_hbm.at[pl.ds(base + c * CHUNK * d, CHUNK * d)], buf, sem
            ).wait()
            if c + 1 < nchunk:
                pltpu.make_async_copy(
                    flat_hbm.at[pl.ds(base + (c + 1) * CHUNK * d, CHUNK * d)],
                    bufs[(c + 1) % 2],
                    sems[(c + 1) % 2],
                ).start()
            for s in range(b):
                lo = jnp.maximum(st_vec[s] - c * CHUNK, 0)
                hi = jnp.minimum(en_vec[s] - c * CHUNK, CHUNK)
                accs = tuple(
                    acc_v[pl.ds(s * d + LANES * j, LANES)] for j in range(d // LANES)
                )

                def rbody(r, a, buf=buf):
                    off = r * d
                    return tuple(
                        jnp.maximum(aj, buf[pl.ds(off + LANES * j, LANES)])
                        for j, aj in enumerate(a)
                    )

                accs = lax.fori_loop(lo, hi, rbody, accs)
                for j in range(d // LANES):
                    acc_v[pl.ds(s * d + LANES * j, LANES)] = accs[j]

        pltpu.sync_copy(acc_v, out_hbm.at[pl.ds(wid * b * d, b * d)])

    return k(flat1d, starts, ends)


def _tc_stage2(partials, b, d):
    def body(p_ref, o_ref):
        acc = p_ref[0:b, :]
        for w in range(1, NW):
            acc = jnp.maximum(acc, p_ref[w * b : (w + 1) * b, :])
        o_ref[...] = acc

    return pl.pallas_call(
        body,
        out_shape=jax.ShapeDtypeStruct((b, d), jnp.float32),
    )(partials)


def kernel(flat, cu_seqlens):
    n, d = flat.shape
    b = cu_seqlens.shape[0] - 1
    rows_w = n // NW
    assert n % NW == 0 and rows_w % CHUNK == 0 and d % LANES == 0

    w = jnp.arange(NW, dtype=jnp.int32)[:, None] * rows_w
    cu = cu_seqlens.astype(jnp.int32)
    starts = (jnp.clip(cu[None, :-1], w, w + rows_w) - w).reshape(-1)
    ends = (jnp.clip(cu[None, 1:], w, w + rows_w) - w).reshape(-1)

    partials = _sc_stage1(flat.reshape(-1), starts, ends, n, d, b)
    return _tc_stage2(partials.reshape(NW * b, d), b, d)
